# baseline (speedup 1.0000x reference)
import functools

import jax
import jax.numpy as jnp
import numpy as np
from jax import lax
from jax.experimental import pallas as pl
from jax.experimental.pallas import tpu as pltpu
from jax.experimental.pallas import tpu_sc as plsc

_SHUFFLE_SEED = 42
_B, _N, _D = 4, 4096, 2048
_ROWS = _B * _N
_NC, _NS = 2, 16
_NW = _NC * _NS
_RPW = _ROWS // _NW
_NSEM = 8

_mesh = plsc.VectorSubcoreMesh(core_axis_name="c", subcore_axis_name="s",
                               num_cores=_NC, num_subcores=_NS)


@functools.partial(
    pl.kernel,
    out_type=jax.ShapeDtypeStruct((_ROWS, _D), jnp.float32),
    mesh=_mesh,
    scratch_types=(
        [pltpu.VMEM((_RPW,), jnp.int32)]
        + [pltpu.SemaphoreType.DMA for _ in range(_NSEM)]
    ),
)
def _sc_shuffle(table_hbm, idx_hbm, out_hbm, idx_v, *sems):
    wid = lax.axis_index("s") * _NC + lax.axis_index("c")
    base = wid * _RPW

    pltpu.sync_copy(idx_hbm.at[pl.ds(base, _RPW)], idx_v)

    def body(g, carry):
        src = idx_v[pl.ds(g, 1)][0]
        pltpu.make_async_copy(
            table_hbm.at[pl.ds(src, 1)],
            out_hbm.at[pl.ds(base + g, 1)],
            sems[0]).start()
        return carry

    lax.fori_loop(0, _RPW, body, 0)
    def drain(g, carry):
        pltpu.make_async_copy(
            table_hbm.at[pl.ds(0, 1)], out_hbm.at[pl.ds(base, 1)],
            sems[0]).wait()
        return carry
    lax.fori_loop(0, _RPW, drain, 0)


def _traced_global_index():
    perm = jax.random.permutation(jax.random.key(_SHUFFLE_SEED), _N)
    return (jnp.arange(_B, dtype=jnp.int32)[:, None] * _N
            + perm[None, :].astype(jnp.int32)).reshape(-1)


def _eager_global_index():
    with jax.default_device(jax.devices("cpu")[0]):
        return np.asarray(_traced_global_index())


try:
    _GLOBAL_IDX = _eager_global_index()
except Exception:
    _GLOBAL_IDX = None


def kernel(inputs):
    flat = inputs.reshape(_ROWS, _D)
    idx = (jnp.asarray(_GLOBAL_IDX) if _GLOBAL_IDX is not None
           else _traced_global_index())
    out = _sc_shuffle(flat, idx)
    return out.reshape(_B, _N, _D)


# speedup vs baseline: 35.8352x; 35.8352x over previous
import functools

import jax
import jax.numpy as jnp
import numpy as np
from jax import lax
from jax.experimental import pallas as pl
from jax.experimental.pallas import tpu as pltpu
from jax.experimental.pallas import tpu_sc as plsc

_SHUFFLE_SEED = 42
_B, _N, _D = 4, 4096, 2048
_ROWS = _B * _N
_NC, _NS = 2, 16
_NW = _NC * _NS
_RPW = _ROWS // _NW
_K = 16
_CHUNKS = _RPW // _K
_NBUF = 3
_LEAD = 2

_mesh = plsc.VectorSubcoreMesh(core_axis_name="c", subcore_axis_name="s",
                               num_cores=_NC, num_subcores=_NS)


@functools.partial(
    pl.kernel,
    out_type=jax.ShapeDtypeStruct((_ROWS, _D), jnp.float32),
    mesh=_mesh,
    scratch_types=(
        [pltpu.VMEM((_RPW,), jnp.int32)]
        + [pltpu.VMEM((_K, _D), jnp.float32) for _ in range(_NBUF)]
        + [pltpu.SemaphoreType.DMA for _ in range(2 * _NBUF)]
    ),
)
def _sc_shuffle(table_hbm, idx_hbm, out_hbm, idx_v, *bufs_and_sems):
    bufs = bufs_and_sems[:_NBUF]
    gsems = bufs_and_sems[_NBUF:2 * _NBUF]
    osems = bufs_and_sems[2 * _NBUF:]
    wid = lax.axis_index("s") * _NC + lax.axis_index("c")
    base = wid * _RPW

    pltpu.sync_copy(idx_hbm.at[pl.ds(base, _RPW)], idx_v)

    def gather_desc(g):
        s = g % _NBUF
        return pltpu.make_async_copy(
            table_hbm.at[idx_v.at[pl.ds(g * _K, _K)]], bufs[s], gsems[s])

    def out_desc(g):
        s = g % _NBUF
        return pltpu.make_async_copy(bufs[s], out_hbm.at[pl.ds(base + g * _K, _K)],
                                     osems[s])

    for g in range(min(_LEAD, _CHUNKS)):
        gather_desc(g).start()
    for g in range(_CHUNKS):
        ahead = g + _LEAD
        if ahead < _CHUNKS:
            if ahead >= _NBUF:
                out_desc(ahead - _NBUF).wait()
            gather_desc(ahead).start()
        gather_desc(g).wait()
        out_desc(g).start()
    for g in range(max(0, _CHUNKS - _NBUF), _CHUNKS):
        out_desc(g).wait()


def _traced_global_index():
    perm = jax.random.permutation(jax.random.key(_SHUFFLE_SEED), _N)
    return (jnp.arange(_B, dtype=jnp.int32)[:, None] * _N
            + perm[None, :].astype(jnp.int32)).reshape(-1)


def _eager_global_index():
    with jax.default_device(jax.devices("cpu")[0]):
        return np.asarray(_traced_global_index())


try:
    _GLOBAL_IDX = _eager_global_index()
except Exception:
    _GLOBAL_IDX = None


def kernel(inputs):
    flat = inputs.reshape(_ROWS, _D)
    idx = (jnp.asarray(_GLOBAL_IDX) if _GLOBAL_IDX is not None
           else _traced_global_index())
    out = _sc_shuffle(flat, idx)
    return out.reshape(_B, _N, _D)
